# Initial kernel scaffold; baseline (speedup 1.0000x reference)
import jax
import jax.numpy as jnp
from jax.experimental import pallas as pl


def kernel(seq, seq_emb, U):
    raise NotImplementedError("write your pallas kernel here")


# speedup vs baseline: 14.6278x; 14.6278x over previous
import functools

import jax
import jax.numpy as jnp
from jax import lax
from jax.experimental import pallas as pl
from jax.experimental.pallas import tpu as pltpu
from jax.experimental.pallas import tpu_sc as plsc

VOCAB = 100000
EMB = 32
RS0 = 7
RADIUS = RS0 // 2
BATCH = 1024
SEQ = 200
OUT_C = 96

NW = 32
ROWS_PER_W = BATCH // NW
CHUNK = 40
NCHUNK = SEQ // CHUNK
HALO = 2 * RADIUS
ROW_LEN = RS0 * EMB


def _sc_body(seq_hbm, emb_hbm, u_hbm, out_hbm, idx_v, rows_v, emb_row, out_v, sem):
    c_id = lax.axis_index("c")
    s_id = lax.axis_index("s")
    wid = s_id * 2 + c_id
    base_row = wid * ROWS_PER_W

    zero = jnp.zeros((16,), jnp.float32)
    for i in range(RADIUS * EMB // 16):
        idx_v
        emb_row[pl.ds(i * 16, 16)] = zero
    for i in range((SEQ + RADIUS) * EMB // 16, (SEQ + HALO) * EMB // 16):
        emb_row[pl.ds(i * 16, 16)] = zero

    def row_body(r, carry):
        row = base_row + r
        pltpu.sync_copy(
            emb_hbm.at[pl.ds(row * (SEQ * EMB), SEQ * EMB)],
            emb_row.at[pl.ds(RADIUS * EMB, SEQ * EMB)],
        )

        def chunk_body(c, carry2):
            l0 = c * CHUNK
            pltpu.sync_copy(seq_hbm.at[pl.ds(row * SEQ + l0, CHUNK)], idx_v)
            pltpu.async_copy(u_hbm.at[idx_v], rows_v, sem).wait()

            def token_body(t, carry3):
                for h in range(2):
                    e = [
                        emb_row[pl.ds((l0 + t + j) * EMB + h * 16, 16)]
                        for j in range(RS0)
                    ]
                    u = [rows_v[t, pl.ds(j * EMB + h * 16, 16)] for j in range(RS0)]
                    p = [e[j] * u[j] for j in range(RS0)]
                    m3 = jnp.maximum(p[2], jnp.maximum(p[3], p[4]))
                    m5 = jnp.maximum(m3, jnp.maximum(p[1], p[5]))
                    m7 = jnp.maximum(m5, jnp.maximum(p[0], p[6]))
                    out_v[pl.ds(t * OUT_C + h * 16, 16)] = m7
                    out_v[pl.ds(t * OUT_C + EMB + h * 16, 16)] = m5
                    out_v[pl.ds(t * OUT_C + 2 * EMB + h * 16, 16)] = m3
                return carry3

            lax.fori_loop(0, CHUNK, token_body, 0)
            pltpu.sync_copy(
                out_v,
                out_hbm.at[pl.ds((row * SEQ + l0) * OUT_C, CHUNK * OUT_C)],
            )
            return carry2

        lax.fori_loop(0, NCHUNK, chunk_body, 0)
        return carry

    lax.fori_loop(0, ROWS_PER_W, row_body, 0)


_sc_kernel = functools.partial(
    pl.kernel,
    mesh=plsc.VectorSubcoreMesh(core_axis_name="c", subcore_axis_name="s"),
    compiler_params=pltpu.CompilerParams(use_tc_tiling_on_sc=False),
    out_type=jax.ShapeDtypeStruct((BATCH * SEQ * OUT_C,), jnp.float32),
    scratch_types=[
        pltpu.VMEM((CHUNK,), jnp.int32),
        pltpu.VMEM((CHUNK, ROW_LEN), jnp.float32),
        pltpu.VMEM(((SEQ + HALO) * EMB,), jnp.float32),
        pltpu.VMEM((CHUNK * OUT_C,), jnp.float32),
        pltpu.SemaphoreType.DMA,
    ],
)(_sc_body)


def kernel(seq, seq_emb, U):
    out_flat = _sc_kernel(
        seq.reshape(BATCH * SEQ),
        seq_emb.reshape(BATCH * SEQ * EMB),
        U.reshape(VOCAB, ROW_LEN),
    )
    return out_flat.reshape(BATCH, SEQ, OUT_C)


# speedup vs baseline: 19.4787x; 1.3316x over previous
import functools

import jax
import jax.numpy as jnp
from jax import lax
from jax.experimental import pallas as pl
from jax.experimental.pallas import tpu as pltpu
from jax.experimental.pallas import tpu_sc as plsc

VOCAB = 100000
EMB = 32
RS0 = 7
RADIUS = RS0 // 2
BATCH = 1024
SEQ = 200
OUT_C = 96

NW = 32
ROWS_PER_W = BATCH // NW
CHUNK = 40
NCHUNK = SEQ // CHUNK
ROW_LEN = RS0 * EMB
TOK_W = ROWS_PER_W * SEQ
NCH_W = TOK_W // CHUNK
EROW = SEQ * EMB
EBUF = EROW + 2 * RADIUS * EMB + 64
OUT_CH = CHUNK * OUT_C


def _sc_body(seq_hbm, emb_hbm, u_hbm, out_hbm, idx_all, ebuf, gbuf, obuf,
             gsem0, gsem1, esem0, esem1, osem0, osem1):
    c_id = lax.axis_index("c")
    s_id = lax.axis_index("s")
    wid = s_id * 2 + c_id
    tok0 = wid * TOK_W
    out0 = tok0 * OUT_C
    gsem = (gsem0, gsem1)
    osem = (osem0, osem1)
    esem = (esem0, esem1)

    zero = jnp.zeros((16,), jnp.float32)
    for b in range(4):
        for i in range(RADIUS * EMB // 16):
            ebuf[b, pl.ds(i * 16, 16)] = zero
            ebuf[b, pl.ds((RADIUS + SEQ) * EMB + i * 16, 16)] = zero

    def gather_desc(ci, par):
        return pltpu.make_async_copy(
            u_hbm.at[idx_all.at[pl.ds(ci * CHUNK, CHUNK)]],
            gbuf.at[par], gsem[par])

    def emb_desc(row, b, par):
        return pltpu.make_async_copy(
            emb_hbm.at[pl.ds(row * EROW, EROW)],
            ebuf.at[b, pl.ds(RADIUS * EMB, EROW)], esem[par])

    def out_desc(ci, par):
        return pltpu.make_async_copy(
            obuf.at[par],
            out_hbm.at[pl.ds(out0 + ci * OUT_CH, OUT_CH)], osem[par])

    def compute_chunk(ci, b, l0, g, o):
        w0 = [ebuf[b, pl.ds((l0 + j) * EMB + h * 16, 16)]
              for j in range(RS0) for h in range(2)]

        def tbody(t, w):
            for h in range(2):
                p = [w[2 * j + h] * gbuf[g, t, pl.ds(j * EMB + h * 16, 16)]
                     for j in range(RS0)]
                m3 = jnp.maximum(p[2], jnp.maximum(p[3], p[4]))
                m5 = jnp.maximum(m3, jnp.maximum(p[1], p[5]))
                m7 = jnp.maximum(m5, jnp.maximum(p[0], p[6]))
                obuf[o, pl.ds(t * OUT_C + h * 16, 16)] = m7
                obuf[o, pl.ds(t * OUT_C + EMB + h * 16, 16)] = m5
                obuf[o, pl.ds(t * OUT_C + 2 * EMB + h * 16, 16)] = m3
            nxt = [ebuf[b, pl.ds((l0 + RS0 + t) * EMB + h * 16, 16)]
                   for h in range(2)]
            return tuple(w[2:]) + tuple(nxt)

        lax.fori_loop(0, CHUNK, tbody, tuple(w0))

    pltpu.sync_copy(seq_hbm.at[pl.ds(tok0, TOK_W)], idx_all)
    base_row = wid * ROWS_PER_W
    emb_desc(base_row + 0, 0, 0).start()
    emb_desc(base_row + 1, 1, 0).start()
    gather_desc(0, 0).start()

    def qbody(q, carry):
        for s in range(2):
            rp = 2 * q + s
            row0 = base_row + 2 * rp
            if s == 0:
                emb_desc(row0 + 2, 2, 1).start()
                emb_desc(row0 + 3, 3, 1).start()
            else:
                @pl.when(q < 7)
                def _():
                    emb_desc(row0 + 2, 0, 0).start()
                    emb_desc(row0 + 3, 1, 0).start()
            emb_desc(row0, 2 * s, s).wait()
            emb_desc(row0 + 1, 2 * s + 1, s).wait()
            for k in range(10):
                ci = rp * 10 + k
                rr = k // 5
                l0 = (k % 5) * CHUNK
                par = k % 2
                npar = (k + 1) % 2
                if s == 1 and k == 9:
                    @pl.when(q < 7)
                    def _():
                        gather_desc(ci + 1, npar).start()
                else:
                    gather_desc(ci + 1, npar).start()
                gather_desc(ci, par).wait()
                if k < 2 and s == 0:
                    @pl.when(q > 0)
                    def _():
                        out_desc(ci - 2, par).wait()
                else:
                    out_desc(ci - 2, par).wait()
                compute_chunk(ci, 2 * s + rr, l0, par, par)
                out_desc(ci, par).start()
        return carry

    lax.fori_loop(0, 8, qbody, 0)
    out_desc(NCH_W - 2, 0).wait()
    out_desc(NCH_W - 1, 1).wait()


_sc_kernel = functools.partial(
    pl.kernel,
    mesh=plsc.VectorSubcoreMesh(core_axis_name="c", subcore_axis_name="s"),
    compiler_params=pltpu.CompilerParams(use_tc_tiling_on_sc=False),
    out_type=jax.ShapeDtypeStruct((BATCH * SEQ * OUT_C,), jnp.float32),
    scratch_types=[
        pltpu.VMEM((TOK_W,), jnp.int32),
        pltpu.VMEM((4, EBUF), jnp.float32),
        pltpu.VMEM((2, CHUNK, ROW_LEN), jnp.float32),
        pltpu.VMEM((2, OUT_CH), jnp.float32),
        pltpu.SemaphoreType.DMA,
        pltpu.SemaphoreType.DMA,
        pltpu.SemaphoreType.DMA,
        pltpu.SemaphoreType.DMA,
        pltpu.SemaphoreType.DMA,
        pltpu.SemaphoreType.DMA,
    ],
)(_sc_body)


def kernel(seq, seq_emb, U):
    out_flat = _sc_kernel(
        seq.reshape(BATCH * SEQ),
        seq_emb.reshape(BATCH * SEQ * EMB),
        U.reshape(VOCAB, ROW_LEN),
    )
    return out_flat.reshape(BATCH, SEQ, OUT_C)
